# baseline (speedup 1.0000x reference)
import functools

import jax
import jax.numpy as jnp
from jax import lax
from jax.experimental import pallas as pl
from jax.experimental.pallas import tpu as pltpu
from jax.experimental.pallas import tpu_sc as plsc

EMB_DIM = 64
CHUNK = 128


@functools.cache
def _make_lookup(n_idx: int):
    info = plsc.get_sparse_core_info()
    nc, ns = info.num_cores, info.num_subcores
    nw = nc * ns
    per_w = n_idx // nw
    n_chunks = per_w // CHUNK
    assert per_w * nw == n_idx and n_chunks * CHUNK == per_w

    mesh = plsc.VectorSubcoreMesh(core_axis_name="c", subcore_axis_name="s")

    @functools.partial(
        pl.kernel,
        mesh=mesh,
        out_type=jax.ShapeDtypeStruct((n_idx, EMB_DIM), jnp.float32),
        scratch_types=[
            pltpu.VMEM((n_chunks, CHUNK), jnp.int32),
            pltpu.VMEM((CHUNK, EMB_DIM), jnp.float32),
            pltpu.SemaphoreType.DMA,
        ],
        compiler_params=pltpu.CompilerParams(use_tc_tiling_on_sc=False),
    )
    def lookup(idx_hbm, table_hbm, out_hbm, idx_v, rows_v, gsem):
        wid = lax.axis_index("s") * nc + lax.axis_index("c")
        base = wid * per_w
        pltpu.sync_copy(idx_hbm.at[wid], idx_v)

        def body(j, carry):
            pltpu.async_copy(table_hbm.at[idx_v.at[j]], rows_v, gsem).wait()
            pltpu.sync_copy(
                rows_v, out_hbm.at[pl.ds(base + j * CHUNK, CHUNK)]
            )
            return carry

        lax.fori_loop(0, n_chunks, body, 0, unroll=False)

    return lookup


def kernel(hidden_state, weight):
    batch, hist = hidden_state.shape
    n_idx = batch * hist
    info = plsc.get_sparse_core_info()
    nw = info.num_cores * info.num_subcores
    idx3d = hidden_state.astype(jnp.int32).reshape(nw, (n_idx // nw) // CHUNK, CHUNK)
    out = _make_lookup(n_idx)(idx3d, weight)
    return out.reshape(batch, hist, EMB_DIM)


# speedup vs baseline: 1.1142x; 1.1142x over previous
import functools

import jax
import jax.numpy as jnp
from jax import lax
from jax.experimental import pallas as pl
from jax.experimental.pallas import tpu as pltpu
from jax.experimental.pallas import tpu_sc as plsc

EMB_DIM = 64
CHUNK = 128
NBUF = 8
K = 4


@functools.cache
def _make_lookup(n_idx: int):
    info = plsc.get_sparse_core_info()
    nc, ns = info.num_cores, info.num_subcores
    nw = nc * ns
    per_w = n_idx // nw
    n_chunks = per_w // CHUNK
    assert per_w * nw == n_idx and n_chunks * CHUNK == per_w
    assert n_chunks % NBUF == 0 and n_chunks >= 2 * NBUF

    mesh = plsc.VectorSubcoreMesh(core_axis_name="c", subcore_axis_name="s")

    @functools.partial(
        pl.kernel,
        mesh=mesh,
        out_type=jax.ShapeDtypeStruct((n_idx, EMB_DIM), jnp.float32),
        scratch_types=[
            pltpu.VMEM((n_chunks, CHUNK), jnp.int32),
            pltpu.VMEM((NBUF, CHUNK, EMB_DIM), jnp.float32),
        ]
        + [pltpu.SemaphoreType.DMA] * (2 * NBUF),
        compiler_params=pltpu.CompilerParams(use_tc_tiling_on_sc=False),
    )
    def lookup(idx_hbm, table_hbm, out_hbm, idx_v, rows_v, *sems):
        gsem = sems[:NBUF]
        ssem = sems[NBUF:]
        wid = lax.axis_index("s") * nc + lax.axis_index("c")
        base = wid * per_w
        pltpu.sync_copy(idx_hbm.at[wid], idx_v)

        def gather_start(j, b):
            pltpu.async_copy(table_hbm.at[idx_v.at[j]], rows_v.at[b], gsem[b])

        def gather_wait(j, b):
            pltpu.make_async_copy(
                table_hbm.at[idx_v.at[j]], rows_v.at[b], gsem[b]
            ).wait()

        def scatter_start(i, b):
            pltpu.async_copy(
                rows_v.at[b],
                out_hbm.at[pl.ds(base + i * CHUNK, CHUNK)],
                ssem[b],
            )

        def scatter_wait(i, b):
            pltpu.make_async_copy(
                rows_v.at[b],
                out_hbm.at[pl.ds(base + i * CHUNK, CHUNK)],
                ssem[b],
            ).wait()

        for j in range(NBUF):
            gather_start(j, j)
            if j >= K:
                i = j - K
                gather_wait(i, i)
                scatter_start(i, i)

        n_blocks = (n_chunks - NBUF) // NBUF

        def blk_body(blk, carry):
            j0 = NBUF + blk * NBUF
            for b in range(NBUF):
                j = j0 + b
                scatter_wait(j - NBUF, b)
                gather_start(j, b)
                bi = (b + NBUF - K) % NBUF
                gather_wait(j - K, bi)
                scatter_start(j - K, bi)
            return carry

        lax.fori_loop(0, n_blocks, blk_body, 0, unroll=False)

        for j in range(n_chunks - K, n_chunks):
            b = j % NBUF
            gather_wait(j, b)
            scatter_start(j, b)
        for b in range(NBUF):
            scatter_wait(n_chunks - NBUF + b, b)

    return lookup


def kernel(hidden_state, weight):
    batch, hist = hidden_state.shape
    n_idx = batch * hist
    info = plsc.get_sparse_core_info()
    nw = info.num_cores * info.num_subcores
    idx3d = hidden_state.astype(jnp.int32).reshape(nw, (n_idx // nw) // CHUNK, CHUNK)
    out = _make_lookup(n_idx)(idx3d, weight)
    return out.reshape(batch, hist, EMB_DIM)
